# baseline (speedup 1.0000x reference)
import jax
import jax.numpy as jnp
from jax.experimental import pallas as pl
from jax.experimental.pallas import tpu as pltpu

_GRID = 1


def _copy_kernel(in_ref, out_ref):
    out_ref[...] = in_ref[...]


def kernel(inputs_embeds, embed_table):
    del embed_table
    b, s, h = inputs_embeds.shape
    rows = s // _GRID
    return pl.pallas_call(
        _copy_kernel,
        grid=(_GRID,),
        in_specs=[pl.BlockSpec((b, rows, h), lambda i: (0, i, 0))],
        out_specs=pl.BlockSpec((b, rows, h), lambda i: (0, i, 0)),
        out_shape=jax.ShapeDtypeStruct((b, s, h), inputs_embeds.dtype),
    )(inputs_embeds)
